# Initial kernel scaffold; baseline (speedup 1.0000x reference)
import jax
import jax.numpy as jnp
from jax.experimental import pallas as pl


def kernel(x, edge_index, W1, b1, Wr1, br1, g1, be1, W2, b2, Wr2, br2, g2, be2, w_att, b_att):
    raise NotImplementedError("write your pallas kernel here")


# speedup vs baseline: 4.7087x; 4.7087x over previous
import functools

import jax
import jax.numpy as jnp
from jax import lax
from jax.experimental import pallas as pl
from jax.experimental.pallas import tpu as pltpu
from jax.experimental.pallas import tpu_sc as plsc

N = 10000
E = 320000
H = 128

NC = 2
NS = 16
NW = NC * NS
EPW = E // NW
CH = 80
NCHUNK = EPW // CH
NPAD = 10240
ROWS_PT = NPAD // NS

_sc_mesh = plsc.VectorSubcoreMesh(core_axis_name="c", subcore_axis_name="s")


@functools.partial(
    pl.kernel,
    out_type=jax.ShapeDtypeStruct((NC, NPAD, H), jnp.float32),
    mesh=_sc_mesh,
    scratch_types=[
        pltpu.VMEM((CH,), jnp.int32),
        pltpu.VMEM((CH,), jnp.int32),
        pltpu.VMEM((CH, H), jnp.float32),
        pltpu.VMEM_SHARED((NPAD, H), jnp.float32),
        pltpu.SemaphoreType.DMA,
    ],
)
def _segsum(h_hbm, src_hbm, dst_hbm, zero_hbm, out_hbm,
            src_v, dst_v, rows_v, acc_sh, sem):
    c = lax.axis_index("c")
    s = lax.axis_index("s")
    wid = c * NS + s

    r0 = s * ROWS_PT
    pltpu.sync_copy(zero_hbm.at[pl.ds(r0, ROWS_PT)], acc_sh.at[pl.ds(r0, ROWS_PT)])
    plsc.subcore_barrier()

    ebase = wid * EPW

    def step(i, carry):
        b = ebase + i * CH
        pltpu.sync_copy(src_hbm.at[pl.ds(b, CH)], src_v)
        pltpu.sync_copy(dst_hbm.at[pl.ds(b, CH)], dst_v)
        pltpu.async_copy(h_hbm.at[src_v], rows_v, sem).wait()
        pltpu.sync_copy(rows_v, acc_sh.at[dst_v], add=True)
        return carry

    lax.fori_loop(0, NCHUNK, step, 0)

    plsc.subcore_barrier()
    pltpu.sync_copy(acc_sh.at[pl.ds(r0, ROWS_PT)],
                    out_hbm.at[c, pl.ds(r0, ROWS_PT)])


def _layer_body(aggp_ref, h_ref, W_ref, b_ref, Wr_ref, br_ref, g_ref, be_ref,
                out_ref):
    agg = aggp_ref[0, :N, :] + aggp_ref[1, :N, :]
    o = jnp.dot(agg, W_ref[...], preferred_element_type=jnp.float32)
    o = jnp.maximum(o + b_ref[...], 0.0)
    r = jnp.dot(h_ref[...], Wr_ref[...], preferred_element_type=jnp.float32)
    r = jnp.maximum(r + br_ref[...], 0.0)
    o = o + r
    mu = jnp.mean(o, axis=0, keepdims=True)
    var = jnp.mean((o - mu) ** 2, axis=0, keepdims=True)
    out_ref[...] = g_ref[...] * (o - mu) / jnp.sqrt(var + 1e-5) + be_ref[...]


_layer = pl.pallas_call(
    _layer_body,
    out_shape=jax.ShapeDtypeStruct((N, H), jnp.float32),
)


def _layer2_readout_body(aggp_ref, h_ref, W_ref, b_ref, Wr_ref, br_ref,
                         g_ref, be_ref, watt_ref, batt_ref, out_ref):
    agg = aggp_ref[0, :N, :] + aggp_ref[1, :N, :]
    o = jnp.dot(agg, W_ref[...], preferred_element_type=jnp.float32)
    o = jnp.maximum(o + b_ref[...], 0.0)
    r = jnp.dot(h_ref[...], Wr_ref[...], preferred_element_type=jnp.float32)
    r = jnp.maximum(r + br_ref[...], 0.0)
    o = o + r
    mu = jnp.mean(o, axis=0, keepdims=True)
    var = jnp.mean((o - mu) ** 2, axis=0, keepdims=True)
    h2 = g_ref[...] * (o - mu) / jnp.sqrt(var + 1e-5) + be_ref[...]
    s = jnp.sum(h2 * watt_ref[...], axis=1, keepdims=True) + batt_ref[...]
    w = jax.nn.sigmoid(s)
    out_ref[:, :H] = jnp.sum(w * h2, axis=0, keepdims=True)
    out_ref[:, H:] = jnp.max(h2, axis=0, keepdims=True)


_layer2_readout = pl.pallas_call(
    _layer2_readout_body,
    out_shape=jax.ShapeDtypeStruct((1, 2 * H), jnp.float32),
)


def kernel(x, edge_index, W1, b1, Wr1, br1, g1, be1,
           W2, b2, Wr2, br2, g2, be2, w_att, b_att):
    src = edge_index[0]
    dst = edge_index[1]
    zeros = jnp.zeros((NPAD, H), jnp.float32)

    aggp1 = _segsum(x, src, dst, zeros)
    h1 = _layer(aggp1, x, W1, b1.reshape(1, H), Wr1, br1.reshape(1, H),
                g1.reshape(1, H), be1.reshape(1, H))
    aggp2 = _segsum(h1, src, dst, zeros)
    out = _layer2_readout(aggp2, h1, W2, b2.reshape(1, H), Wr2,
                          br2.reshape(1, H), g2.reshape(1, H),
                          be2.reshape(1, H), w_att.reshape(1, H),
                          b_att.reshape(1, 1))
    return out
